# baseline (speedup 1.0000x reference)
import functools

import jax
import jax.numpy as jnp
from jax import lax
from jax.experimental import pallas as pl
from jax.experimental.pallas import tpu as pltpu
from jax.experimental.pallas import tpu_sc as plsc

DIM = 64
LANES = 16
NUM_CORES = 2
NUM_SUBCORES = 16
NUM_WORKERS = NUM_CORES * NUM_SUBCORES
BB = 128
SUP = 8
SC_ROWS = 2
CHUNK = SC_ROWS * BB


def _emb_fused(ids_t, sids_t, word, seg, batch, seq):
  n_sup = seq // SUP
  nbb = batch // BB
  mesh = plsc.VectorSubcoreMesh(
      core_axis_name="c", subcore_axis_name="s",
      num_cores=NUM_CORES, num_subcores=NUM_SUBCORES)

  @functools.partial(
      pl.kernel,
      out_type=jax.ShapeDtypeStruct((seq, DIM // 8, nbb, 8, BB), jnp.float32),
      mesh=mesh,
      scratch_types=[
          pltpu.VMEM((SUP, BB), jnp.int32),
          pltpu.VMEM((SUP, BB), jnp.int32),
          pltpu.VMEM((CHUNK, DIM), jnp.float32),
          pltpu.VMEM((SC_ROWS, DIM // 8, 1, 8, BB), jnp.float32),
          pltpu.VMEM((2, DIM), jnp.float32),
          pltpu.SemaphoreType.DMA,
      ],
      compiler_params=pltpu.CompilerParams(
          use_tc_tiling_on_sc=False, needs_layout_passes=False),
  )
  def body(ids_hbm, sids_hbm, word_hbm, seg_hbm, out_hbm,
           idx_b, sid_b, rows_v, out_v, seg_v, gsem):
    wid = lax.axis_index("s") * NUM_CORES + lax.axis_index("c")
    b0 = wid * BB
    pltpu.sync_copy(seg_hbm, seg_v)
    s0 = [seg_v[0, pl.ds(LANES * j, LANES)] for j in range(DIM // LANES)]
    sd = [seg_v[1, pl.ds(LANES * j, LANES)] - a for j, a in enumerate(s0)]
    lanes_iota = lax.iota(jnp.int32, LANES)
    dt_vecs = [2 * j + lanes_iota // 8 for j in range(DIM // LANES)]
    dr_vec = lanes_iota % 8
    zeros = jnp.zeros((LANES,), jnp.int32)

    def sup_body(p, carry):
      srow0 = p * SUP
      pltpu.sync_copy(ids_hbm.at[pl.ds(srow0, SUP), pl.ds(b0, BB)], idx_b)
      pltpu.sync_copy(sids_hbm.at[pl.ds(srow0, SUP), pl.ds(b0, BB)], sid_b)

      for q in range(SUP // SC_ROWS):
        cps = [
            pltpu.async_copy(
                word_hbm.at[idx_b.at[q * SC_ROWS + ri]],
                rows_v.at[pl.ds(ri * BB, BB)], gsem)
            for ri in range(SC_ROWS)
        ]
        for cp in cps:
          cp.wait()

        for ri in range(SC_ROWS):
          rr = q * SC_ROWS + ri
          i_ri = jnp.full((LANES,), ri, jnp.int32)

          def group_body(g, c2, rr=rr, ri=ri, i_ri=i_ri):
            go = g * LANES
            sv = sid_b[rr, pl.ds(go, LANES)].astype(jnp.float32)
            for i in range(LANES):
              tok = ri * BB + go + i
              fv = jnp.full((LANES,), sv[i], jnp.float32)
              i_b = jnp.full((LANES,), go + i, jnp.int32)
              for j in range(DIM // LANES):
                val = (rows_v[tok, pl.ds(LANES * j, LANES)]
                       + (s0[j] + fv * sd[j]))
                plsc.store_scatter(
                    out_v, [i_ri, dt_vecs[j], zeros, dr_vec, i_b], val)
            return c2

          lax.fori_loop(0, BB // LANES, group_body, 0)

        pltpu.sync_copy(
            out_v,
            out_hbm.at[pl.ds(srow0 + q * SC_ROWS, SC_ROWS), :,
                       pl.ds(wid, 1), :, :])
      return carry

    lax.fori_loop(0, n_sup, sup_body, 0)

  return body(ids_t, sids_t, word, seg)


def kernel(input_ids, segment_ids, word_embeddings, segments_embedding):
  b, s = input_ids.shape
  ids_t = jnp.transpose(input_ids)
  sids_t = jnp.transpose(segment_ids)
  out5 = _emb_fused(ids_t, sids_t, word_embeddings, segments_embedding, b, s)
  out = jnp.transpose(out5, (2, 4, 0, 1, 3)).reshape(b, s, DIM)
  return (out, None)
